# Initial kernel scaffold; baseline (speedup 1.0000x reference)
import functools

import jax
import jax.numpy as jnp
from jax import lax
from jax.experimental import pallas as pl
from jax.experimental.pallas import tpu as pltpu
from jax.experimental.pallas import tpu_sc as plsc


def _make_gather(N, V, D, n_workers, chunk):
    nchunk = N // (n_workers * chunk)
    b_per_w = N // n_workers
    mesh = plsc.VectorSubcoreMesh(core_axis_name="c", subcore_axis_name="s")
    info = plsc.get_sparse_core_info()
    nc = info.num_cores

    @functools.partial(
        pl.kernel,
        mesh=mesh,
        out_type=jax.ShapeDtypeStruct((N, D), jnp.float32),
        scratch_types=[
            pltpu.VMEM((chunk,), jnp.int32),
            pltpu.VMEM((chunk, D), jnp.float32),
            pltpu.SemaphoreType.DMA,
        ],
    )
    def gather_kernel(table_hbm, idx_hbm, out_hbm, idx_v, rows_v, sem):
        wid = lax.axis_index("s") * nc + lax.axis_index("c")
        base = wid * b_per_w

        def body(j, carry):
            off = base + j * chunk
            pltpu.sync_copy(idx_hbm.at[pl.ds(off, chunk)], idx_v)
            pltpu.async_copy(table_hbm.at[idx_v], rows_v, sem).wait()
            pltpu.sync_copy(rows_v, out_hbm.at[pl.ds(off, chunk)])
            return carry

        lax.fori_loop(0, nchunk, body, 0)

    return gather_kernel


def kernel(word_indexes, W):
    B, L = word_indexes.shape
    V, D = W.shape
    N = B * L
    idx = word_indexes.reshape(N).astype(jnp.int32)
    out = _make_gather(N, V, D, n_workers=32, chunk=1024)(W, idx)
    return out.reshape(B, L, D)


# speedup vs baseline: 1.5108x; 1.5108x over previous
import functools

import jax
import jax.numpy as jnp
from jax import lax
from jax.experimental import pallas as pl
from jax.experimental.pallas import tpu as pltpu
from jax.experimental.pallas import tpu_sc as plsc


def _make_gather(N, V, D, n_workers, chunk, nbuf=3):
    nchunk = N // (n_workers * chunk)
    b_per_w = N // n_workers
    mesh = plsc.VectorSubcoreMesh(core_axis_name="c", subcore_axis_name="s")
    info = plsc.get_sparse_core_info()
    nc = info.num_cores

    @functools.partial(
        pl.kernel,
        mesh=mesh,
        out_type=jax.ShapeDtypeStruct((N, D), jnp.float32),
        scratch_types=[
            pltpu.VMEM((nchunk, chunk), jnp.int32),
            pltpu.VMEM((nbuf, chunk, D), jnp.float32),
            pltpu.SemaphoreType.DMA((nchunk,)),
            pltpu.SemaphoreType.DMA((nbuf,)),
            pltpu.SemaphoreType.DMA((nbuf,)),
        ],
        compiler_params=pltpu.CompilerParams(use_tc_tiling_on_sc=False),
    )
    def gather_kernel(table_hbm, idx_hbm, out_hbm, idx_v, rows_v, idx_sem,
                      gat_sem, out_sem):
        wid = lax.axis_index("s") * nc + lax.axis_index("c")
        base = wid * b_per_w

        idx_cps = []
        for j in range(nchunk):
            cp = pltpu.make_async_copy(
                idx_hbm.at[pl.ds(base + j * chunk, chunk)], idx_v.at[j],
                idx_sem.at[j])
            cp.start()
            idx_cps.append(cp)

        gat_cps = [None] * nbuf
        out_cps = [None] * nbuf
        for j in range(nchunk):
            slot = j % nbuf
            if out_cps[slot] is not None:
                out_cps[slot].wait()
                out_cps[slot] = None
            idx_cps[j].wait()
            cp = pltpu.make_async_copy(
                table_hbm.at[idx_v.at[j]], rows_v.at[slot], gat_sem.at[slot])
            cp.start()
            gat_cps[slot] = cp
            prev = (j - 1) % nbuf
            if j >= 1 and gat_cps[prev] is not None:
                gat_cps[prev].wait()
                gat_cps[prev] = None
                ocp = pltpu.make_async_copy(
                    rows_v.at[prev],
                    out_hbm.at[pl.ds(base + (j - 1) * chunk, chunk)],
                    out_sem.at[prev])
                ocp.start()
                out_cps[prev] = ocp

        last = (nchunk - 1) % nbuf
        gat_cps[last].wait()
        ocp = pltpu.make_async_copy(
            rows_v.at[last],
            out_hbm.at[pl.ds(base + (nchunk - 1) * chunk, chunk)],
            out_sem.at[last])
        ocp.start()
        out_cps[last] = ocp
        for cp in out_cps:
            if cp is not None:
                cp.wait()

    return gather_kernel


def kernel(word_indexes, W):
    B, L = word_indexes.shape
    V, D = W.shape
    N = B * L
    idx = word_indexes.reshape(N).astype(jnp.int32)
    out = _make_gather(N, V, D, n_workers=32, chunk=1024)(W, idx)
    return out.reshape(B, L, D)
